# Initial kernel scaffold; baseline (speedup 1.0000x reference)
import functools

import jax
import jax.numpy as jnp
from jax import lax
from jax.experimental import pallas as pl
from jax.experimental.pallas import tpu as pltpu
from jax.experimental.pallas import tpu_sc as plsc

NC = 2
NS = 16
L = 16
CH = 80


def _prep(x, prompt, W, a_src, a_dst):
    N, D = x.shape
    B = 1000
    grid = (N // B,)

    def body(x_ref, p_ref, w_ref, as_ref, ad_ref,
             hw_ref, es_ref, ed_ref, s_ref, smax_ref):
        i = pl.program_id(0)
        h = x_ref[...] + p_ref[...]
        hw = jnp.dot(h, w_ref[...], preferred_element_type=jnp.float32)
        hw_ref[...] = hw
        es = jnp.sum(hw * as_ref[...][None, :], axis=1)
        ed = jnp.sum(hw * ad_ref[...][None, :], axis=1)
        es_ref[...] = es
        ed_ref[...] = ed
        bm = jnp.max(es)

        @pl.when(i == 0)
        def _():
            smax_ref[0] = bm

        @pl.when(i > 0)
        def _():
            smax_ref[0] = jnp.maximum(smax_ref[0], bm)

        s_ref[...] = jnp.full((L,), smax_ref[0], jnp.float32)

    return pl.pallas_call(
        body,
        grid=grid,
        in_specs=[
            pl.BlockSpec((B, D), lambda i: (i, 0)),
            pl.BlockSpec((B, D), lambda i: (i, 0)),
            pl.BlockSpec((D, D), lambda i: (0, 0)),
            pl.BlockSpec((D,), lambda i: (0,)),
            pl.BlockSpec((D,), lambda i: (0,)),
        ],
        out_specs=[
            pl.BlockSpec((B, D), lambda i: (i, 0)),
            pl.BlockSpec((B,), lambda i: (i,)),
            pl.BlockSpec((B,), lambda i: (i,)),
            pl.BlockSpec((L,), lambda i: (0,)),
        ],
        out_shape=[
            jax.ShapeDtypeStruct((N, D), jnp.float32),
            jax.ShapeDtypeStruct((N,), jnp.float32),
            jax.ShapeDtypeStruct((N,), jnp.float32),
            jax.ShapeDtypeStruct((L,), jnp.float32),
        ],
        scratch_shapes=[pltpu.SMEM((1,), jnp.float32)],
    )(x, prompt, W, a_src, a_dst)


def _gat_sc(src, dst, e_src, e_dst, s16, hW):
    N, D = hW.shape
    E = src.shape[0]
    EA = E // NS
    EB = E // (NC * NS)
    SEG = 640
    SEG_LAST = N - SEG * (NS - 1)

    mesh = plsc.VectorSubcoreMesh(core_axis_name="c", subcore_axis_name="s")

    @functools.partial(
        pl.kernel,
        out_type=jax.ShapeDtypeStruct((NC, N, D), jnp.float32),
        mesh=mesh,
        scratch_types=[
            pltpu.VMEM((EA,), jnp.int32),
            pltpu.VMEM((EA,), jnp.int32),
            pltpu.VMEM((N,), jnp.float32),
            pltpu.VMEM((N,), jnp.float32),
            pltpu.VMEM((N,), jnp.float32),
            pltpu.VMEM((N,), jnp.float32),
            pltpu.VMEM((L,), jnp.float32),
            pltpu.VMEM((CH,), jnp.float32),
            pltpu.VMEM((CH,), jnp.float32),
            pltpu.VMEM((CH,), jnp.int32),
            pltpu.VMEM((CH, D), jnp.float32),
            pltpu.VMEM((SEG,), jnp.float32),
            pltpu.VMEM((CH, D), jnp.float32),
            pltpu.VMEM_SHARED((N,), jnp.float32),
            pltpu.VMEM_SHARED((N, D), jnp.float32),
        ],
    )
    def k(src_hbm, dst_hbm, es_hbm, ed_hbm, s_hbm, hw_hbm, out_hbm,
          sidx, didx, es_tab, ed_tab, c_tab, dn_tab, svec, exbuf, alpha,
          didx_chunk, rows, zvec, zrows, denom_sh, acc_sh):
        cid = lax.axis_index("c")
        sid = lax.axis_index("s")

        pltpu.sync_copy(es_hbm, es_tab)
        pltpu.sync_copy(ed_hbm, ed_tab)
        pltpu.sync_copy(s_hbm, svec)
        sv = svec[...]

        @pl.loop(0, N, step=L)
        def _(i):
            z = sv + ed_tab[pl.ds(i, L)]
            c_tab[pl.ds(i, L)] = jnp.maximum(z, 0.2 * z)

        @pl.loop(0, SEG, step=L)
        def _(i):
            zvec[pl.ds(i, L)] = jnp.zeros((L,), jnp.float32)

        @pl.loop(0, CH)
        def _(r):
            for j in range(D // L):
                zrows[r, pl.ds(j * L, L)] = jnp.zeros((L,), jnp.float32)

        start = sid * SEG

        @pl.when(sid < NS - 1)
        def _():
            pltpu.sync_copy(zvec, denom_sh.at[pl.ds(start, SEG)])
            for j in range(SEG // CH):
                pltpu.sync_copy(zrows, acc_sh.at[pl.ds(start + j * CH, CH)])

        @pl.when(sid == NS - 1)
        def _():
            pltpu.sync_copy(zvec.at[pl.ds(0, SEG_LAST)],
                            denom_sh.at[pl.ds(start, SEG_LAST)])
            for j in range(SEG_LAST // CH):
                pltpu.sync_copy(zrows, acc_sh.at[pl.ds(start + j * CH, CH)])

        base_a = sid * EA
        pltpu.sync_copy(src_hbm.at[pl.ds(base_a, EA)], sidx)
        pltpu.sync_copy(dst_hbm.at[pl.ds(base_a, EA)], didx)

        plsc.subcore_barrier()

        @pl.loop(0, EA, step=CH)
        def _(ch):
            ch = pl.multiple_of(ch, 8)
            for j in range(CH // L):
                off = ch + j * L
                svi = sidx[pl.ds(off, L)]
                dvi = didx[pl.ds(off, L)]
                es = plsc.load_gather(es_tab, [svi])
                ed = plsc.load_gather(ed_tab, [dvi])
                cv = plsc.load_gather(c_tab, [dvi])
                t = es + ed
                e = jnp.maximum(t, 0.2 * t)
                exbuf[pl.ds(j * L, L)] = jnp.exp(e - cv)
                didx_chunk[pl.ds(j * L, L)] = dvi
            pltpu.sync_copy(exbuf, denom_sh.at[didx_chunk], add=True)

        plsc.subcore_barrier()

        pltpu.sync_copy(denom_sh, dn_tab)
        wid = cid * NS + sid
        base_b = wid * EB
        pltpu.sync_copy(src_hbm.at[pl.ds(base_b, EB)], sidx.at[pl.ds(0, EB)])
        pltpu.sync_copy(dst_hbm.at[pl.ds(base_b, EB)], didx.at[pl.ds(0, EB)])

        @pl.loop(0, EB, step=CH)
        def _(ch):
            ch = pl.multiple_of(ch, 8)
            pltpu.sync_copy(hw_hbm.at[sidx.at[pl.ds(ch, CH)]], rows)
            for j in range(CH // L):
                off = ch + j * L
                svi = sidx[pl.ds(off, L)]
                dvi = didx[pl.ds(off, L)]
                es = plsc.load_gather(es_tab, [svi])
                ed = plsc.load_gather(ed_tab, [dvi])
                cv = plsc.load_gather(c_tab, [dvi])
                dn = plsc.load_gather(dn_tab, [dvi])
                t = es + ed
                e = jnp.maximum(t, 0.2 * t)
                ex = jnp.exp(e - cv)
                alpha[pl.ds(j * L, L)] = ex / (dn + 1e-16)
                didx_chunk[pl.ds(j * L, L)] = dvi

            @pl.loop(0, CH)
            def _(kk):
                a = plsc.load_gather(alpha, [jnp.full((L,), kk, jnp.int32)])
                for j in range(D // L):
                    rows[kk, pl.ds(j * L, L)] = rows[kk, pl.ds(j * L, L)] * a

            pltpu.sync_copy(rows, acc_sh.at[didx_chunk], add=True)

        plsc.subcore_barrier()

        @pl.when(sid < NS - 1)
        def _():
            pltpu.sync_copy(acc_sh.at[pl.ds(start, SEG)],
                            out_hbm.at[cid, pl.ds(start, SEG)])

        @pl.when(sid == NS - 1)
        def _():
            pltpu.sync_copy(acc_sh.at[pl.ds(start, SEG_LAST)],
                            out_hbm.at[cid, pl.ds(start, SEG_LAST)])

    return k(src, dst, e_src, e_dst, s16, hW)


def _final_add(part):
    _, N, D = part.shape
    B = 1000

    def body(p_ref, o_ref):
        o_ref[...] = p_ref[0] + p_ref[1]

    return pl.pallas_call(
        body,
        grid=(N // B,),
        in_specs=[pl.BlockSpec((2, B, D), lambda i: (0, i, 0))],
        out_specs=pl.BlockSpec((B, D), lambda i: (i, 0)),
        out_shape=jax.ShapeDtypeStruct((N, D), jnp.float32),
    )(part)


def kernel(x, prompt, edge_index, W, a_src, a_dst):
    hW, e_src, e_dst, s16 = _prep(x, prompt, W, a_src, a_dst)
    part = _gat_sc(edge_index[0], edge_index[1], e_src, e_dst, s16, hW)
    return _final_add(part)


# speedup vs baseline: 19.7981x; 19.7981x over previous
import dataclasses
import functools

import jax
import jax.numpy as jnp
from jax import lax
from jax.experimental import pallas as pl
from jax.experimental.pallas import tpu as pltpu
from jax.experimental.pallas import tpu_sc as plsc

NC = 2
NS = 16
L = 16
CH = 80


def _prep(x, prompt, W, a_src, a_dst):
    N, D = x.shape
    B = 1000
    grid = (N // B,)

    def body(x_ref, p_ref, w_ref, as_ref, ad_ref,
             hw_ref, es_ref, ed_ref, s_ref, smax_ref):
        i = pl.program_id(0)
        h = x_ref[...] + p_ref[...]
        hw = jnp.dot(h, w_ref[...], preferred_element_type=jnp.float32)
        hw_ref[...] = hw
        es = jnp.sum(hw * as_ref[...][None, :], axis=1)
        ed = jnp.sum(hw * ad_ref[...][None, :], axis=1)
        es_ref[...] = es[None, None, :]
        ed_ref[...] = ed[None, None, :]
        bm = jnp.max(es)

        @pl.when(i == 0)
        def _():
            smax_ref[0] = bm

        @pl.when(i > 0)
        def _():
            smax_ref[0] = jnp.maximum(smax_ref[0], bm)

        s_ref[...] = jnp.full((L,), smax_ref[0], jnp.float32)

    return pl.pallas_call(
        body,
        grid=grid,
        in_specs=[
            pl.BlockSpec((B, D), lambda i: (i, 0)),
            pl.BlockSpec((B, D), lambda i: (i, 0)),
            pl.BlockSpec((D, D), lambda i: (0, 0)),
            pl.BlockSpec((D,), lambda i: (0,)),
            pl.BlockSpec((D,), lambda i: (0,)),
        ],
        out_specs=[
            pl.BlockSpec((B, D), lambda i: (i, 0)),
            pl.BlockSpec((1, 1, B), lambda i: (i, 0, 0)),
            pl.BlockSpec((1, 1, B), lambda i: (i, 0, 0)),
            pl.BlockSpec((L,), lambda i: (0,)),
        ],
        out_shape=[
            jax.ShapeDtypeStruct((N, D), jnp.float32),
            jax.ShapeDtypeStruct((N // B, 1, B), jnp.float32),
            jax.ShapeDtypeStruct((N // B, 1, B), jnp.float32),
            jax.ShapeDtypeStruct((L,), jnp.float32),
        ],
        scratch_shapes=[pltpu.SMEM((1,), jnp.float32)],
    )(x, prompt, W, a_src, a_dst)


def _gat_sc(src, dst, e_src, e_dst, s16, hW):
    N, D = hW.shape
    E = src.shape[0]
    EA = E // NS
    EB = E // (NC * NS)
    BL = 2000
    SEG = 640
    SEG_LAST = N - SEG * (NS - 1)

    mesh = plsc.VectorSubcoreMesh(core_axis_name="c", subcore_axis_name="s")
    cp = pltpu.CompilerParams()
    if "needs_layout_passes" in pltpu.CompilerParams.__dataclass_fields__:
        cp = dataclasses.replace(cp, needs_layout_passes=False)

    @functools.partial(
        pl.kernel,
        out_type=jax.ShapeDtypeStruct((NC, N, D), jnp.float32),
        mesh=mesh,
        compiler_params=cp,
        scratch_types=[
            pltpu.VMEM((BL,), jnp.int32),
            pltpu.VMEM((BL,), jnp.int32),
            pltpu.VMEM((N,), jnp.float32),
            pltpu.VMEM((N,), jnp.float32),
            pltpu.VMEM((N,), jnp.float32),
            pltpu.VMEM((L,), jnp.float32),
            pltpu.VMEM((CH,), jnp.float32),
            pltpu.VMEM((CH,), jnp.float32),
            pltpu.VMEM((CH,), jnp.int32),
            pltpu.VMEM((CH, D), jnp.float32),
            pltpu.VMEM((SEG,), jnp.float32),
            pltpu.VMEM_SHARED((N,), jnp.float32),
            pltpu.VMEM_SHARED((N, D), jnp.float32),
        ],
    )
    def k(src_hbm, dst_hbm, es_hbm, ed_hbm, s_hbm, hw_hbm, out_hbm,
          sidx, didx, es_tab, ed_tab, dn_tab, svec, exbuf, alpha,
          didx_chunk, rows, zvec, denom_sh, acc_sh):
        cid = lax.axis_index("c")
        sid = lax.axis_index("s")

        pltpu.sync_copy(es_hbm, es_tab)
        pltpu.sync_copy(ed_hbm, ed_tab)
        pltpu.sync_copy(s_hbm, svec)
        sv = svec[...]

        @pl.loop(0, SEG, step=L)
        def _(i):
            zvec[pl.ds(i, L)] = jnp.zeros((L,), jnp.float32)

        @pl.loop(0, CH)
        def _(r):
            for j in range(D // L):
                rows[r, pl.ds(j * L, L)] = jnp.zeros((L,), jnp.float32)

        start = sid * SEG

        @pl.when(sid < NS - 1)
        def _():
            pltpu.sync_copy(zvec, denom_sh.at[pl.ds(start, SEG)])
            for j in range(SEG // CH):
                pltpu.sync_copy(rows, acc_sh.at[pl.ds(start + j * CH, CH)])

        @pl.when(sid == NS - 1)
        def _():
            pltpu.sync_copy(zvec.at[pl.ds(0, SEG_LAST)],
                            denom_sh.at[pl.ds(start, SEG_LAST)])
            for j in range(SEG_LAST // CH):
                pltpu.sync_copy(rows, acc_sh.at[pl.ds(start + j * CH, CH)])

        plsc.subcore_barrier()

        base_a = sid * EA

        @pl.loop(0, EA, step=BL)
        def _(blk):
            boff = pl.multiple_of(base_a + blk, 8)
            pltpu.sync_copy(src_hbm.at[pl.ds(boff, BL)], sidx)
            pltpu.sync_copy(dst_hbm.at[pl.ds(boff, BL)], didx)

            @pl.loop(0, BL, step=CH)
            def _(ch):
                ch = pl.multiple_of(ch, 8)
                for j in range(CH // L):
                    off = ch + j * L
                    svi = sidx[pl.ds(off, L)]
                    dvi = didx[pl.ds(off, L)]
                    es = plsc.load_gather(es_tab, [svi])
                    ed = plsc.load_gather(ed_tab, [dvi])
                    z = sv + ed
                    cv = jnp.maximum(z, 0.2 * z)
                    t = es + ed
                    e = jnp.maximum(t, 0.2 * t)
                    exbuf[pl.ds(j * L, L)] = jnp.exp(e - cv)
                    didx_chunk[pl.ds(j * L, L)] = dvi
                pltpu.sync_copy(exbuf, denom_sh.at[didx_chunk], add=True)

        plsc.subcore_barrier()

        pltpu.sync_copy(denom_sh, dn_tab)
        wid = cid * NS + sid
        base_b = wid * EB

        @pl.loop(0, EB, step=BL)
        def _(blk):
            boff = pl.multiple_of(base_b + blk, 8)
            pltpu.sync_copy(src_hbm.at[pl.ds(boff, BL)], sidx)
            pltpu.sync_copy(dst_hbm.at[pl.ds(boff, BL)], didx)

            @pl.loop(0, BL, step=CH)
            def _(ch):
                ch = pl.multiple_of(ch, 8)
                pltpu.sync_copy(hw_hbm.at[sidx.at[pl.ds(ch, CH)]], rows)
                for j in range(CH // L):
                    off = ch + j * L
                    svi = sidx[pl.ds(off, L)]
                    dvi = didx[pl.ds(off, L)]
                    es = plsc.load_gather(es_tab, [svi])
                    ed = plsc.load_gather(ed_tab, [dvi])
                    dn = plsc.load_gather(dn_tab, [dvi])
                    z = sv + ed
                    cv = jnp.maximum(z, 0.2 * z)
                    t = es + ed
                    e = jnp.maximum(t, 0.2 * t)
                    ex = jnp.exp(e - cv)
                    alpha[pl.ds(j * L, L)] = ex / (dn + 1e-16)
                    didx_chunk[pl.ds(j * L, L)] = dvi

                @pl.loop(0, CH)
                def _(kk):
                    a = plsc.load_gather(alpha, [jnp.full((L,), kk, jnp.int32)])
                    for j in range(D // L):
                        rows[kk, pl.ds(j * L, L)] = rows[kk, pl.ds(j * L, L)] * a

                pltpu.sync_copy(rows, acc_sh.at[didx_chunk], add=True)

        plsc.subcore_barrier()

        @pl.when(sid < NS - 1)
        def _():
            pltpu.sync_copy(acc_sh.at[pl.ds(start, SEG)],
                            out_hbm.at[cid, pl.ds(start, SEG)])

        @pl.when(sid == NS - 1)
        def _():
            pltpu.sync_copy(acc_sh.at[pl.ds(start, SEG_LAST)],
                            out_hbm.at[cid, pl.ds(start, SEG_LAST)])

    return k(src, dst, e_src, e_dst, s16, hW)


def _final_add(part):
    _, N, D = part.shape
    B = 1000

    def body(p_ref, o_ref):
        o_ref[...] = p_ref[0] + p_ref[1]

    return pl.pallas_call(
        body,
        grid=(N // B,),
        in_specs=[pl.BlockSpec((2, B, D), lambda i: (0, i, 0))],
        out_specs=pl.BlockSpec((B, D), lambda i: (i, 0)),
        out_shape=jax.ShapeDtypeStruct((N, D), jnp.float32),
    )(part)


def kernel(x, prompt, edge_index, W, a_src, a_dst):
    hW, e_src2, e_dst2, s16 = _prep(x, prompt, W, a_src, a_dst)
    e_src = e_src2.reshape(-1)
    e_dst = e_dst2.reshape(-1)
    part = _gat_sc(edge_index[0], edge_index[1], e_src, e_dst, s16, hW)
    return _final_add(part)


# speedup vs baseline: 31.8625x; 1.6094x over previous
import dataclasses
import functools

import jax
import jax.numpy as jnp
from jax import lax
from jax.experimental import pallas as pl
from jax.experimental.pallas import tpu as pltpu
from jax.experimental.pallas import tpu_sc as plsc

NC = 2
NS = 16
L = 16
CH = 80


def _prep(x, prompt, W, a_src, a_dst):
    N, D = x.shape
    B = 1000
    grid = (N // B,)

    def body(x_ref, p_ref, w_ref, as_ref, ad_ref,
             hw_ref, es_ref, ed_ref, s_ref, smax_ref):
        i = pl.program_id(0)
        h = x_ref[...] + p_ref[...]
        hw = jnp.dot(h, w_ref[...], preferred_element_type=jnp.float32)
        hw_ref[...] = hw
        es = jnp.sum(hw * as_ref[...][None, :], axis=1)
        ed = jnp.sum(hw * ad_ref[...][None, :], axis=1)
        es_ref[...] = es[None, None, :]
        ed_ref[...] = ed[None, None, :]
        bm = jnp.max(es)

        @pl.when(i == 0)
        def _():
            smax_ref[0] = bm

        @pl.when(i > 0)
        def _():
            smax_ref[0] = jnp.maximum(smax_ref[0], bm)

        s_ref[...] = jnp.full((L,), smax_ref[0], jnp.float32)

    return pl.pallas_call(
        body,
        grid=grid,
        in_specs=[
            pl.BlockSpec((B, D), lambda i: (i, 0)),
            pl.BlockSpec((B, D), lambda i: (i, 0)),
            pl.BlockSpec((D, D), lambda i: (0, 0)),
            pl.BlockSpec((D,), lambda i: (0,)),
            pl.BlockSpec((D,), lambda i: (0,)),
        ],
        out_specs=[
            pl.BlockSpec((B, D), lambda i: (i, 0)),
            pl.BlockSpec((1, 1, B), lambda i: (i, 0, 0)),
            pl.BlockSpec((1, 1, B), lambda i: (i, 0, 0)),
            pl.BlockSpec((L,), lambda i: (0,)),
        ],
        out_shape=[
            jax.ShapeDtypeStruct((N, D), jnp.float32),
            jax.ShapeDtypeStruct((N // B, 1, B), jnp.float32),
            jax.ShapeDtypeStruct((N // B, 1, B), jnp.float32),
            jax.ShapeDtypeStruct((L,), jnp.float32),
        ],
        scratch_shapes=[pltpu.SMEM((1,), jnp.float32)],
    )(x, prompt, W, a_src, a_dst)


def _gat_sc(src, dst, e_src, e_dst, s16, hW):
    N, D = hW.shape
    E = src.shape[0]
    EB = E // (NC * NS)
    BL = 2000
    NCH = BL // CH
    SEG = 640
    SEG_LAST = N - SEG * (NS - 1)
    NP = SEG * NS

    mesh = plsc.VectorSubcoreMesh(core_axis_name="c", subcore_axis_name="s")
    cp = pltpu.CompilerParams()
    if "needs_layout_passes" in pltpu.CompilerParams.__dataclass_fields__:
        cp = dataclasses.replace(cp, needs_layout_passes=False)

    @functools.partial(
        pl.kernel,
        out_type=[
            jax.ShapeDtypeStruct((NC, N, D), jnp.float32),
            jax.ShapeDtypeStruct((NC, 1, NP), jnp.float32),
        ],
        mesh=mesh,
        compiler_params=cp,
        scratch_types=[
            pltpu.VMEM((BL,), jnp.int32),
            pltpu.VMEM((BL,), jnp.int32),
            pltpu.VMEM((N,), jnp.float32),
            pltpu.VMEM((N,), jnp.float32),
            pltpu.VMEM((L,), jnp.float32),
            pltpu.VMEM((CH,), jnp.float32),
            pltpu.VMEM((CH,), jnp.int32),
            pltpu.VMEM((CH, D), jnp.float32),
            pltpu.VMEM((CH, D), jnp.float32),
            pltpu.VMEM((SEG,), jnp.float32),
            pltpu.SemaphoreType.DMA,
            pltpu.SemaphoreType.DMA,
            pltpu.VMEM_SHARED((NP,), jnp.float32),
            pltpu.VMEM_SHARED((N, D), jnp.float32),
        ],
    )
    def k(src_hbm, dst_hbm, es_hbm, ed_hbm, s_hbm, hw_hbm, out_hbm, den_hbm,
          sidx, didx, es_tab, ed_tab, svec, exbuf,
          didx_chunk, rows0, rows1, zvec, sem0, sem1, denom_sh, acc_sh):
        cid = lax.axis_index("c")
        sid = lax.axis_index("s")

        pltpu.sync_copy(es_hbm, es_tab)
        pltpu.sync_copy(ed_hbm, ed_tab)
        pltpu.sync_copy(s_hbm, svec)
        sv = svec[...]

        @pl.loop(0, SEG, step=L)
        def _(i):
            zvec[pl.ds(i, L)] = jnp.zeros((L,), jnp.float32)

        @pl.loop(0, CH)
        def _(r):
            for j in range(D // L):
                rows0[r, pl.ds(j * L, L)] = jnp.zeros((L,), jnp.float32)

        start = sid * SEG

        pltpu.sync_copy(zvec, denom_sh.at[pl.ds(start, SEG)])

        @pl.when(sid < NS - 1)
        def _():
            for j in range(SEG // CH):
                pltpu.sync_copy(rows0, acc_sh.at[pl.ds(start + j * CH, CH)])

        @pl.when(sid == NS - 1)
        def _():
            for j in range(SEG_LAST // CH):
                pltpu.sync_copy(rows0, acc_sh.at[pl.ds(start + j * CH, CH)])

        plsc.subcore_barrier()

        wid = cid * NS + sid
        base = wid * EB

        def gather(ch, buf, sem):
            pltpu.make_async_copy(
                hw_hbm.at[sidx.at[pl.ds(ch, CH)]], buf, sem).start()

        def wait(ch, buf, sem):
            pltpu.make_async_copy(
                hw_hbm.at[sidx.at[pl.ds(ch, CH)]], buf, sem).wait()

        def process(ch, buf, sem):
            wait(ch, buf, sem)
            for j in range(CH // L):
                off = ch + j * L
                svi = sidx[pl.ds(off, L)]
                dvi = didx[pl.ds(off, L)]
                es = plsc.load_gather(es_tab, [svi])
                ed = plsc.load_gather(ed_tab, [dvi])
                z = sv + ed
                cv = jnp.maximum(z, 0.2 * z)
                t = es + ed
                e = jnp.maximum(t, 0.2 * t)
                exbuf[pl.ds(j * L, L)] = jnp.exp(e - cv)
                didx_chunk[pl.ds(j * L, L)] = dvi

            @pl.loop(0, CH)
            def _(kk):
                a = plsc.load_gather(exbuf, [jnp.full((L,), kk, jnp.int32)])
                for j in range(D // L):
                    buf[kk, pl.ds(j * L, L)] = buf[kk, pl.ds(j * L, L)] * a

            pltpu.sync_copy(buf, acc_sh.at[didx_chunk], add=True)
            pltpu.sync_copy(exbuf, denom_sh.at[didx_chunk], add=True)

        @pl.loop(0, EB, step=BL)
        def _(blk):
            boff = pl.multiple_of(base + blk, 8)
            pltpu.sync_copy(src_hbm.at[pl.ds(boff, BL)], sidx)
            pltpu.sync_copy(dst_hbm.at[pl.ds(boff, BL)], didx)

            gather(0, rows0, sem0)

            @pl.loop(0, NCH, step=2)
            def _(q):
                ch0 = pl.multiple_of(q * CH, 8)
                ch1 = ch0 + CH

                @pl.when(ch1 < BL)
                def _():
                    gather(ch1, rows1, sem1)

                process(ch0, rows0, sem0)

                @pl.when(ch1 < BL)
                def _():
                    @pl.when(ch1 + CH < BL)
                    def _():
                        gather(ch1 + CH, rows0, sem0)

                    process(ch1, rows1, sem1)

        plsc.subcore_barrier()

        pltpu.sync_copy(denom_sh.at[pl.ds(start, SEG)],
                        den_hbm.at[cid, 0, pl.ds(start, SEG)])

        @pl.when(sid < NS - 1)
        def _():
            pltpu.sync_copy(acc_sh.at[pl.ds(start, SEG)],
                            out_hbm.at[cid, pl.ds(start, SEG)])

        @pl.when(sid == NS - 1)
        def _():
            pltpu.sync_copy(acc_sh.at[pl.ds(start, SEG_LAST)],
                            out_hbm.at[cid, pl.ds(start, SEG_LAST)])

    return k(src, dst, e_src, e_dst, s16, hW)


def _finalize(part, den):
    _, N, D = part.shape
    B = 1000

    def body(p_ref, d_ref, o_ref):
        dn = d_ref[:, 0] + d_ref[:, 1] + 1e-16
        o_ref[...] = (p_ref[0] + p_ref[1]) / dn[:, None]

    return pl.pallas_call(
        body,
        grid=(N // B,),
        in_specs=[
            pl.BlockSpec((2, B, D), lambda i: (0, i, 0)),
            pl.BlockSpec((B, 2), lambda i: (i, 0)),
        ],
        out_specs=pl.BlockSpec((B, D), lambda i: (i, 0)),
        out_shape=jax.ShapeDtypeStruct((N, D), jnp.float32),
    )(part, den[:, 0, :part.shape[1]].T)


def kernel(x, prompt, edge_index, W, a_src, a_dst):
    hW, e_src2, e_dst2, s16 = _prep(x, prompt, W, a_src, a_dst)
    e_src = e_src2.reshape(-1)
    e_dst = e_dst2.reshape(-1)
    part, den = _gat_sc(edge_index[0], edge_index[1], e_src, e_dst, s16, hW)
    return _finalize(part, den)


# speedup vs baseline: 39.3091x; 1.2337x over previous
import dataclasses
import functools

import jax
import jax.numpy as jnp
from jax import lax
from jax.experimental import pallas as pl
from jax.experimental.pallas import tpu as pltpu
from jax.experimental.pallas import tpu_sc as plsc

NC = 2
NS = 16
L = 16
CH = 80


def _prep(x, prompt, W, a_src, a_dst):
    N, D = x.shape
    B = 1000
    grid = (N // B,)

    def body(x_ref, p_ref, w_ref, as_ref, ad_ref,
             hw_ref, es_ref, ed_ref, s_ref, smax_ref):
        i = pl.program_id(0)
        h = x_ref[...] + p_ref[...]
        hw = jnp.dot(h, w_ref[...], preferred_element_type=jnp.float32)
        hw_ref[...] = hw
        es = jnp.sum(hw * as_ref[...][None, :], axis=1)
        ed = jnp.sum(hw * ad_ref[...][None, :], axis=1)
        es_ref[...] = es[None, None, :]
        ed_ref[...] = ed[None, None, :]
        bm = jnp.max(es)

        @pl.when(i == 0)
        def _():
            smax_ref[0] = bm

        @pl.when(i > 0)
        def _():
            smax_ref[0] = jnp.maximum(smax_ref[0], bm)

        s_ref[...] = jnp.full((L,), smax_ref[0], jnp.float32)

    return pl.pallas_call(
        body,
        grid=grid,
        in_specs=[
            pl.BlockSpec((B, D), lambda i: (i, 0)),
            pl.BlockSpec((B, D), lambda i: (i, 0)),
            pl.BlockSpec((D, D), lambda i: (0, 0)),
            pl.BlockSpec((D,), lambda i: (0,)),
            pl.BlockSpec((D,), lambda i: (0,)),
        ],
        out_specs=[
            pl.BlockSpec((B, D), lambda i: (i, 0)),
            pl.BlockSpec((1, 1, B), lambda i: (i, 0, 0)),
            pl.BlockSpec((1, 1, B), lambda i: (i, 0, 0)),
            pl.BlockSpec((L,), lambda i: (0,)),
        ],
        out_shape=[
            jax.ShapeDtypeStruct((N, D), jnp.float32),
            jax.ShapeDtypeStruct((N // B, 1, B), jnp.float32),
            jax.ShapeDtypeStruct((N // B, 1, B), jnp.float32),
            jax.ShapeDtypeStruct((L,), jnp.float32),
        ],
        scratch_shapes=[pltpu.SMEM((1,), jnp.float32)],
    )(x, prompt, W, a_src, a_dst)


def _gat_sc(src, dst, e_src, e_dst, s16, hW):
    N, D = hW.shape
    E = src.shape[0]
    EB = E // (NC * NS)
    BL = 2000
    NCH = BL // CH
    SEG = 640
    SEG_LAST = N - SEG * (NS - 1)
    NP = SEG * NS

    mesh = plsc.VectorSubcoreMesh(core_axis_name="c", subcore_axis_name="s")
    cp = pltpu.CompilerParams()
    if "needs_layout_passes" in pltpu.CompilerParams.__dataclass_fields__:
        cp = dataclasses.replace(cp, needs_layout_passes=False)

    @functools.partial(
        pl.kernel,
        out_type=[
            jax.ShapeDtypeStruct((NC, N, D), jnp.float32),
            jax.ShapeDtypeStruct((NC, 1, NP), jnp.float32),
        ],
        mesh=mesh,
        compiler_params=cp,
        scratch_types=[
            pltpu.VMEM((BL,), jnp.int32),
            pltpu.VMEM((BL,), jnp.int32),
            pltpu.VMEM((N,), jnp.float32),
            pltpu.VMEM((N,), jnp.float32),
            pltpu.VMEM((L,), jnp.float32),
            pltpu.VMEM((CH,), jnp.float32),
            pltpu.VMEM((CH,), jnp.float32),
            pltpu.VMEM((CH,), jnp.int32),
            pltpu.VMEM((CH,), jnp.int32),
            pltpu.VMEM((CH, D), jnp.float32),
            pltpu.VMEM((CH, D), jnp.float32),
            pltpu.VMEM((SEG,), jnp.float32),
            pltpu.SemaphoreType.DMA,
            pltpu.SemaphoreType.DMA,
            pltpu.SemaphoreType.DMA,
            pltpu.SemaphoreType.DMA,
            pltpu.SemaphoreType.DMA,
            pltpu.SemaphoreType.DMA,
            pltpu.VMEM_SHARED((NP,), jnp.float32),
            pltpu.VMEM_SHARED((N, D), jnp.float32),
        ],
    )
    def k(src_hbm, dst_hbm, es_hbm, ed_hbm, s_hbm, hw_hbm, out_hbm, den_hbm,
          sidx, didx, es_tab, ed_tab, svec, exb0, exb1, dix0, dix1,
          rows0, rows1, zvec, gs0, gs1, rs0, rs1, es0, es1,
          denom_sh, acc_sh):
        cid = lax.axis_index("c")
        sid = lax.axis_index("s")
        SETS = ((rows0, exb0, dix0, gs0, rs0, es0),
                (rows1, exb1, dix1, gs1, rs1, es1))

        pltpu.sync_copy(es_hbm, es_tab)
        pltpu.sync_copy(ed_hbm, ed_tab)
        pltpu.sync_copy(s_hbm, svec)
        sv = svec[...]

        @pl.loop(0, SEG, step=L)
        def _(i):
            zvec[pl.ds(i, L)] = jnp.zeros((L,), jnp.float32)

        for rows, exb, dix, _, _, _ in SETS:
            @pl.loop(0, CH)
            def _(r, rows=rows):
                for j in range(D // L):
                    rows[r, pl.ds(j * L, L)] = jnp.zeros((L,), jnp.float32)

            @pl.loop(0, CH, step=L)
            def _(i, exb=exb, dix=dix):
                exb[pl.ds(i, L)] = jnp.zeros((L,), jnp.float32)
                dix[pl.ds(i, L)] = jnp.zeros((L,), jnp.int32)

        start = sid * SEG

        pltpu.sync_copy(zvec, denom_sh.at[pl.ds(start, SEG)])

        @pl.when(sid < NS - 1)
        def _():
            for j in range(SEG // CH):
                pltpu.sync_copy(rows0, acc_sh.at[pl.ds(start + j * CH, CH)])

        @pl.when(sid == NS - 1)
        def _():
            for j in range(SEG_LAST // CH):
                pltpu.sync_copy(rows0, acc_sh.at[pl.ds(start + j * CH, CH)])

        plsc.subcore_barrier()

        for rows, exb, dix, _, rs, es in SETS:
            pltpu.async_copy(rows, acc_sh.at[dix], rs, add=True)
            pltpu.async_copy(exb, denom_sh.at[dix], es, add=True)

        wid = cid * NS + sid
        base = wid * EB

        def gather(ch, s):
            rows, exb, dix, gs, rs, es = s
            pltpu.make_async_copy(rows, acc_sh.at[dix], rs).wait()
            pltpu.make_async_copy(exb, denom_sh.at[dix], es).wait()
            pltpu.make_async_copy(
                hw_hbm.at[sidx.at[pl.ds(ch, CH)]], rows, gs).start()

        def process(ch, s):
            rows, exb, dix, gs, rs, es = s
            pltpu.make_async_copy(
                hw_hbm.at[sidx.at[pl.ds(ch, CH)]], rows, gs).wait()
            NJ = CH // L
            svis = [sidx[pl.ds(ch + j * L, L)] for j in range(NJ)]
            dvis = [didx[pl.ds(ch + j * L, L)] for j in range(NJ)]
            ess = [plsc.load_gather(es_tab, [svis[j]]) for j in range(NJ)]
            eds = [plsc.load_gather(ed_tab, [dvis[j]]) for j in range(NJ)]
            for j in range(NJ):
                z = sv + eds[j]
                cv = jnp.maximum(z, 0.2 * z)
                t = ess[j] + eds[j]
                e = jnp.maximum(t, 0.2 * t)
                exb[pl.ds(j * L, L)] = jnp.exp(e - cv)
                dix[pl.ds(j * L, L)] = dvis[j]

            @plsc.parallel_loop(0, CH, 1, unroll=4)
            def _(kk):
                a = plsc.load_gather(exb, [jnp.full((L,), kk, jnp.int32)])
                for j in range(D // L):
                    rows[kk, pl.ds(j * L, L)] = rows[kk, pl.ds(j * L, L)] * a

            pltpu.async_copy(rows, acc_sh.at[dix], rs, add=True)
            pltpu.async_copy(exb, denom_sh.at[dix], es, add=True)

        @pl.loop(0, EB, step=BL)
        def _(blk):
            boff = pl.multiple_of(base + blk, 8)
            pltpu.sync_copy(src_hbm.at[pl.ds(boff, BL)], sidx)
            pltpu.sync_copy(dst_hbm.at[pl.ds(boff, BL)], didx)

            gather(0, SETS[0])

            @pl.loop(0, NCH, step=2)
            def _(q):
                ch0 = pl.multiple_of(q * CH, 8)
                ch1 = ch0 + CH

                @pl.when(ch1 < BL)
                def _():
                    gather(ch1, SETS[1])

                process(ch0, SETS[0])

                @pl.when(ch1 < BL)
                def _():
                    @pl.when(ch1 + CH < BL)
                    def _():
                        gather(ch1 + CH, SETS[0])

                    process(ch1, SETS[1])

        for rows, exb, dix, _, rs, es in SETS:
            pltpu.make_async_copy(rows, acc_sh.at[dix], rs).wait()
            pltpu.make_async_copy(exb, denom_sh.at[dix], es).wait()

        plsc.subcore_barrier()

        pltpu.sync_copy(denom_sh.at[pl.ds(start, SEG)],
                        den_hbm.at[cid, 0, pl.ds(start, SEG)])

        @pl.when(sid < NS - 1)
        def _():
            pltpu.sync_copy(acc_sh.at[pl.ds(start, SEG)],
                            out_hbm.at[cid, pl.ds(start, SEG)])

        @pl.when(sid == NS - 1)
        def _():
            pltpu.sync_copy(acc_sh.at[pl.ds(start, SEG_LAST)],
                            out_hbm.at[cid, pl.ds(start, SEG_LAST)])

    return k(src, dst, e_src, e_dst, s16, hW)


def _finalize(part, den):
    _, N, D = part.shape
    B = 1000

    def body(p_ref, d_ref, o_ref):
        dn = d_ref[:, 0] + d_ref[:, 1] + 1e-16
        o_ref[...] = (p_ref[0] + p_ref[1]) / dn[:, None]

    return pl.pallas_call(
        body,
        grid=(N // B,),
        in_specs=[
            pl.BlockSpec((2, B, D), lambda i: (0, i, 0)),
            pl.BlockSpec((B, 2), lambda i: (i, 0)),
        ],
        out_specs=pl.BlockSpec((B, D), lambda i: (i, 0)),
        out_shape=jax.ShapeDtypeStruct((N, D), jnp.float32),
    )(part, den[:, 0, :part.shape[1]].T)


def kernel(x, prompt, edge_index, W, a_src, a_dst):
    hW, e_src2, e_dst2, s16 = _prep(x, prompt, W, a_src, a_dst)
    e_src = e_src2.reshape(-1)
    e_dst = e_dst2.reshape(-1)
    part, den = _gat_sc(edge_index[0], edge_index[1], e_src, e_dst, s16, hW)
    return _finalize(part, den)


# speedup vs baseline: 39.6150x; 1.0078x over previous
import dataclasses
import functools

import jax
import jax.numpy as jnp
from jax import lax
from jax.experimental import pallas as pl
from jax.experimental.pallas import tpu as pltpu
from jax.experimental.pallas import tpu_sc as plsc

NC = 2
NS = 16
L = 16
CH = 80


def _prep(x, prompt, W, a_src, a_dst):
    N, D = x.shape
    B = 1000
    grid = (N // B,)

    def body(x_ref, p_ref, w_ref, as_ref, ad_ref,
             hw_ref, es_ref, ed_ref, s_ref, smax_ref):
        i = pl.program_id(0)
        h = x_ref[...] + p_ref[...]
        hw = jnp.dot(h, w_ref[...], preferred_element_type=jnp.float32)
        hw_ref[...] = hw
        es = jnp.sum(hw * as_ref[...][None, :], axis=1)
        ed = jnp.sum(hw * ad_ref[...][None, :], axis=1)
        es_ref[...] = es[None, None, :]
        ed_ref[...] = ed[None, None, :]
        bm = jnp.max(es)

        @pl.when(i == 0)
        def _():
            smax_ref[0] = bm

        @pl.when(i > 0)
        def _():
            smax_ref[0] = jnp.maximum(smax_ref[0], bm)

        s_ref[...] = jnp.full((L,), smax_ref[0], jnp.float32)

    return pl.pallas_call(
        body,
        grid=grid,
        in_specs=[
            pl.BlockSpec((B, D), lambda i: (i, 0)),
            pl.BlockSpec((B, D), lambda i: (i, 0)),
            pl.BlockSpec((D, D), lambda i: (0, 0)),
            pl.BlockSpec((D,), lambda i: (0,)),
            pl.BlockSpec((D,), lambda i: (0,)),
        ],
        out_specs=[
            pl.BlockSpec((B, D), lambda i: (i, 0)),
            pl.BlockSpec((1, 1, B), lambda i: (i, 0, 0)),
            pl.BlockSpec((1, 1, B), lambda i: (i, 0, 0)),
            pl.BlockSpec((L,), lambda i: (0,)),
        ],
        out_shape=[
            jax.ShapeDtypeStruct((N, D), jnp.float32),
            jax.ShapeDtypeStruct((N // B, 1, B), jnp.float32),
            jax.ShapeDtypeStruct((N // B, 1, B), jnp.float32),
            jax.ShapeDtypeStruct((L,), jnp.float32),
        ],
        scratch_shapes=[pltpu.SMEM((1,), jnp.float32)],
    )(x, prompt, W, a_src, a_dst)


def _gat_sc(src, dst, e_src, e_dst, s16, hW):
    N, D = hW.shape
    E = src.shape[0]
    EB = E // (NC * NS)
    BL = 2000
    NCH = BL // CH
    SEG = 640
    SEG_LAST = N - SEG * (NS - 1)
    NP = SEG * NS

    mesh = plsc.VectorSubcoreMesh(core_axis_name="c", subcore_axis_name="s")
    cp = pltpu.CompilerParams()
    if "needs_layout_passes" in pltpu.CompilerParams.__dataclass_fields__:
        cp = dataclasses.replace(cp, needs_layout_passes=False)

    @functools.partial(
        pl.kernel,
        out_type=[
            jax.ShapeDtypeStruct((NC, N, D), jnp.float32),
            jax.ShapeDtypeStruct((NC, 1, NP), jnp.float32),
        ],
        mesh=mesh,
        compiler_params=cp,
        scratch_types=[
            pltpu.VMEM((BL,), jnp.int32),
            pltpu.VMEM((BL,), jnp.int32),
            pltpu.VMEM((N,), jnp.float32),
            pltpu.VMEM((N,), jnp.float32),
            pltpu.VMEM((L,), jnp.float32),
            pltpu.VMEM((CH,), jnp.float32),
            pltpu.VMEM((CH,), jnp.float32),
            pltpu.VMEM((CH,), jnp.int32),
            pltpu.VMEM((CH,), jnp.int32),
            pltpu.VMEM((CH, D), jnp.float32),
            pltpu.VMEM((CH, D), jnp.float32),
            pltpu.VMEM((SEG,), jnp.float32),
            pltpu.SemaphoreType.DMA,
            pltpu.SemaphoreType.DMA,
            pltpu.SemaphoreType.DMA,
            pltpu.SemaphoreType.DMA,
            pltpu.SemaphoreType.DMA,
            pltpu.SemaphoreType.DMA,
            pltpu.VMEM_SHARED((NP,), jnp.float32),
            pltpu.VMEM_SHARED((N, D), jnp.float32),
        ],
    )
    def k(src_hbm, dst_hbm, es_hbm, ed_hbm, s_hbm, hw_hbm, out_hbm, den_hbm,
          sidx, didx, es_tab, ed_tab, svec, exb0, exb1, dix0, dix1,
          rows0, rows1, zvec, gs0, gs1, rs0, rs1, es0, es1,
          denom_sh, acc_sh):
        cid = lax.axis_index("c")
        sid = lax.axis_index("s")
        SETS = ((rows0, exb0, dix0, gs0, rs0, es0),
                (rows1, exb1, dix1, gs1, rs1, es1))

        pltpu.sync_copy(es_hbm, es_tab)
        pltpu.sync_copy(ed_hbm, ed_tab)
        pltpu.sync_copy(s_hbm, svec)
        sv = svec[...]

        @pl.loop(0, SEG, step=L)
        def _(i):
            zvec[pl.ds(i, L)] = jnp.zeros((L,), jnp.float32)

        for rows, exb, dix, _, _, _ in SETS:
            @pl.loop(0, CH)
            def _(r, rows=rows):
                for j in range(D // L):
                    rows[r, pl.ds(j * L, L)] = jnp.zeros((L,), jnp.float32)

            @pl.loop(0, CH, step=L)
            def _(i, exb=exb, dix=dix):
                exb[pl.ds(i, L)] = jnp.zeros((L,), jnp.float32)
                dix[pl.ds(i, L)] = jnp.zeros((L,), jnp.int32)

        start = sid * SEG

        pltpu.sync_copy(zvec, denom_sh.at[pl.ds(start, SEG)])

        @pl.when(sid < NS - 1)
        def _():
            for j in range(SEG // CH):
                pltpu.sync_copy(rows0, acc_sh.at[pl.ds(start + j * CH, CH)])

        @pl.when(sid == NS - 1)
        def _():
            for j in range(SEG_LAST // CH):
                pltpu.sync_copy(rows0, acc_sh.at[pl.ds(start + j * CH, CH)])

        plsc.subcore_barrier()

        for rows, exb, dix, _, rs, es in SETS:
            pltpu.async_copy(rows, acc_sh.at[dix], rs, add=True)
            pltpu.async_copy(exb, denom_sh.at[dix], es, add=True)

        wid = cid * NS + sid
        base = wid * EB

        def gather(ch, s):
            rows, exb, dix, gs, rs, es = s
            pltpu.make_async_copy(rows, acc_sh.at[dix], rs).wait()
            pltpu.make_async_copy(exb, denom_sh.at[dix], es).wait()
            pltpu.make_async_copy(
                hw_hbm.at[sidx.at[pl.ds(ch, CH)]], rows, gs).start()

        def process(ch, s):
            rows, exb, dix, gs, rs, es = s
            NJ = CH // L
            svis = [sidx[pl.ds(ch + j * L, L)] for j in range(NJ)]
            dvis = [didx[pl.ds(ch + j * L, L)] for j in range(NJ)]
            ess = [plsc.load_gather(es_tab, [svis[j]]) for j in range(NJ)]
            eds = [plsc.load_gather(ed_tab, [dvis[j]]) for j in range(NJ)]
            for j in range(NJ):
                z = sv + eds[j]
                cv = jnp.maximum(z, 0.2 * z)
                t = ess[j] + eds[j]
                e = jnp.maximum(t, 0.2 * t)
                exb[pl.ds(j * L, L)] = jnp.exp(e - cv)
                dix[pl.ds(j * L, L)] = dvis[j]

            pltpu.make_async_copy(
                hw_hbm.at[sidx.at[pl.ds(ch, CH)]], rows, gs).wait()

            @plsc.parallel_loop(0, CH, 1, unroll=4)
            def _(kk):
                a = plsc.load_gather(exb, [jnp.full((L,), kk, jnp.int32)])
                for j in range(D // L):
                    rows[kk, pl.ds(j * L, L)] = rows[kk, pl.ds(j * L, L)] * a

            pltpu.async_copy(rows, acc_sh.at[dix], rs, add=True)
            pltpu.async_copy(exb, denom_sh.at[dix], es, add=True)

        @pl.loop(0, EB, step=BL)
        def _(blk):
            boff = pl.multiple_of(base + blk, 8)
            pltpu.sync_copy(src_hbm.at[pl.ds(boff, BL)], sidx)
            pltpu.sync_copy(dst_hbm.at[pl.ds(boff, BL)], didx)

            gather(0, SETS[0])

            @pl.loop(0, NCH, step=2)
            def _(q):
                ch0 = pl.multiple_of(q * CH, 8)
                ch1 = ch0 + CH

                @pl.when(ch1 < BL)
                def _():
                    gather(ch1, SETS[1])

                process(ch0, SETS[0])

                @pl.when(ch1 < BL)
                def _():
                    @pl.when(ch1 + CH < BL)
                    def _():
                        gather(ch1 + CH, SETS[0])

                    process(ch1, SETS[1])

        for rows, exb, dix, _, rs, es in SETS:
            pltpu.make_async_copy(rows, acc_sh.at[dix], rs).wait()
            pltpu.make_async_copy(exb, denom_sh.at[dix], es).wait()

        plsc.subcore_barrier()

        pltpu.sync_copy(denom_sh.at[pl.ds(start, SEG)],
                        den_hbm.at[cid, 0, pl.ds(start, SEG)])

        @pl.when(sid < NS - 1)
        def _():
            pltpu.sync_copy(acc_sh.at[pl.ds(start, SEG)],
                            out_hbm.at[cid, pl.ds(start, SEG)])

        @pl.when(sid == NS - 1)
        def _():
            pltpu.sync_copy(acc_sh.at[pl.ds(start, SEG_LAST)],
                            out_hbm.at[cid, pl.ds(start, SEG_LAST)])

    return k(src, dst, e_src, e_dst, s16, hW)


def _finalize(part, den):
    _, N, D = part.shape
    B = 1000

    def body(p_ref, d_ref, o_ref):
        dn = d_ref[:, 0] + d_ref[:, 1] + 1e-16
        o_ref[...] = (p_ref[0] + p_ref[1]) / dn[:, None]

    return pl.pallas_call(
        body,
        grid=(N // B,),
        in_specs=[
            pl.BlockSpec((2, B, D), lambda i: (0, i, 0)),
            pl.BlockSpec((B, 2), lambda i: (i, 0)),
        ],
        out_specs=pl.BlockSpec((B, D), lambda i: (i, 0)),
        out_shape=jax.ShapeDtypeStruct((N, D), jnp.float32),
    )(part, den[:, 0, :part.shape[1]].T)


def kernel(x, prompt, edge_index, W, a_src, a_dst):
    hW, e_src2, e_dst2, s16 = _prep(x, prompt, W, a_src, a_dst)
    e_src = e_src2.reshape(-1)
    e_dst = e_dst2.reshape(-1)
    part, den = _gat_sc(edge_index[0], edge_index[1], e_src, e_dst, s16, hW)
    return _finalize(part, den)
